# baseline (speedup 1.0000x reference)
import functools

import numpy as np
import jax
import jax.numpy as jnp
from jax.experimental import pallas as pl
from jax.experimental.pallas import tpu as pltpu


_VMEM_LIMIT_BYTES = 48 * 1024 * 1024


def _round_up(n, m):
    return ((n + m - 1) // m) * m


@functools.lru_cache(maxsize=None)
def _selection_matrix(hw):
    j = np.arange(hw * 128, dtype=np.int64)
    sel = (j[:, None] // hw == np.arange(128, dtype=np.int64)[None, :])
    return (sel.astype(np.float32) / np.float32(hw))


def _pool_linear_kernel(x_ref, s_ref, w_ref, b_ref, o_ref, *, tb, c):
    p = jnp.dot(x_ref[...], s_ref[...], preferred_element_type=jnp.float32)
    pooled = p.reshape(tb, c)
    y = jnp.dot(pooled, w_ref[...], preferred_element_type=jnp.float32)
    o_ref[...] = (y + b_ref[...]).astype(o_ref.dtype)


def _choose_tb(batch, row_bytes, g):
    budget = 8 * 1024 * 1024
    cap = max(1, budget // (row_bytes * g))
    for tb in (128, 64, 32, 16, 8, 4, 2, 1):
        if tb <= cap and batch % tb == 0 and batch // tb >= 2:
            return tb
    return 1


def kernel(x, weight_t, bias2):
    B, C, H, W = x.shape
    HW = H * W
    G = C // 128
    ROW = 128 * HW
    Lp = weight_t.shape[1]
    n_label = 1000

    x2 = x.reshape(B * G, ROW)
    sel = jnp.asarray(_selection_matrix(HW))

    tb = _choose_tb(B, ROW * x.dtype.itemsize, G)
    grid = (B // tb,)

    cost = pl.CostEstimate(
        flops=int(2 * B * G * ROW * 128 + 2 * B * C * Lp),
        transcendentals=0,
        bytes_accessed=int(x.dtype.itemsize * B * C * HW
                           + 4 * (ROW * 128 + C * Lp + Lp + B * Lp)))

    out = pl.pallas_call(
        functools.partial(_pool_linear_kernel, tb=tb, c=C),
        out_shape=jax.ShapeDtypeStruct((B, Lp), jnp.float32),
        grid=grid,
        in_specs=[
            pl.BlockSpec((tb * G, ROW), lambda i: (i, 0)),
            pl.BlockSpec((ROW, 128), lambda i: (0, 0)),
            pl.BlockSpec((C, Lp), lambda i: (0, 0)),
            pl.BlockSpec((1, Lp), lambda i: (0, 0)),
        ],
        out_specs=pl.BlockSpec((tb, Lp), lambda i: (i, 0)),
        compiler_params=pltpu.CompilerParams(
            dimension_semantics=("parallel",),
            vmem_limit_bytes=_VMEM_LIMIT_BYTES),
        cost_estimate=cost,
    )(x2, sel, weight_t, bias2)

    return out[:, :n_label]


# speedup vs baseline: 3.4533x; 3.4533x over previous
import functools

import numpy as np
import jax
import jax.numpy as jnp
from jax.experimental import pallas as pl
from jax.experimental.pallas import tpu as pltpu


_VMEM_LIMIT_BYTES = 48 * 1024 * 1024


def _round_up(n, m):
    return ((n + m - 1) // m) * m


@functools.lru_cache(maxsize=None)
def _selection_matrix(hw):
    j = np.arange(hw * 128, dtype=np.int64)
    sel = (j[:, None] // hw == np.arange(128, dtype=np.int64)[None, :])
    return (sel.astype(np.float32) / np.float32(hw))


def _pool_linear_kernel(x_ref, s_ref, w_ref, b_ref, o_ref, *, g, row):
    s = s_ref[...]
    pooled = jnp.concatenate(
        [jnp.dot(x_ref[:, i * row:(i + 1) * row], s,
                 preferred_element_type=jnp.float32) for i in range(g)],
        axis=1)
    y = jnp.dot(pooled, w_ref[...], preferred_element_type=jnp.float32)
    o_ref[...] = (y + b_ref[...]).astype(o_ref.dtype)


def _choose_tb(batch, row_bytes, g):
    budget = 8 * 1024 * 1024
    cap = max(1, budget // (row_bytes * g))
    for tb in (128, 64, 32, 16, 8, 4, 2, 1):
        if tb <= cap and batch % tb == 0 and batch // tb >= 2:
            return tb
    return 1


def kernel(x, weight_t, bias2):
    B, C, H, W = x.shape
    HW = H * W
    G = C // 128
    ROW = 128 * HW
    Lp = weight_t.shape[1]
    n_label = 1000

    x2 = x.reshape(B, C * HW)
    sel = jnp.asarray(_selection_matrix(HW))

    tb = _choose_tb(B, ROW * x.dtype.itemsize, G)
    grid = (B // tb,)

    cost = pl.CostEstimate(
        flops=int(2 * B * G * ROW * 128 + 2 * B * C * Lp),
        transcendentals=0,
        bytes_accessed=int(x.dtype.itemsize * B * C * HW
                           + 4 * (ROW * 128 + C * Lp + Lp + B * Lp)))

    out = pl.pallas_call(
        functools.partial(_pool_linear_kernel, g=G, row=ROW),
        out_shape=jax.ShapeDtypeStruct((B, Lp), jnp.float32),
        grid=grid,
        in_specs=[
            pl.BlockSpec((tb, C * HW), lambda i: (i, 0)),
            pl.BlockSpec((ROW, 128), lambda i: (0, 0)),
            pl.BlockSpec((C, Lp), lambda i: (0, 0)),
            pl.BlockSpec((1, Lp), lambda i: (0, 0)),
        ],
        out_specs=pl.BlockSpec((tb, Lp), lambda i: (i, 0)),
        compiler_params=pltpu.CompilerParams(
            dimension_semantics=("parallel",),
            vmem_limit_bytes=_VMEM_LIMIT_BYTES),
        cost_estimate=cost,
    )(x2, sel, weight_t, bias2)

    return out[:, :n_label]


# speedup vs baseline: 29.1938x; 8.4538x over previous
import functools

import jax
import jax.numpy as jnp
from jax.experimental import pallas as pl
from jax.experimental.pallas import tpu as pltpu


_VMEM_LIMIT_BYTES = 48 * 1024 * 1024


def _pool_linear_kernel(x_ref, w_ref, b_ref, o_ref, *, inv_hw):
    pooled = jnp.sum(x_ref[...], axis=0) * inv_hw
    y = jnp.dot(pooled, w_ref[...], preferred_element_type=jnp.float32)
    o_ref[...] = (y + b_ref[...]).astype(o_ref.dtype)


def _choose_tb(batch, hw, c, itemsize):
    budget = 8 * 1024 * 1024
    cap = max(1, budget // (hw * c * itemsize))
    for tb in (256, 128, 64, 32, 16, 8, 4, 2, 1):
        if tb <= cap and batch % tb == 0 and batch // tb >= 2:
            return tb
    return 1


def kernel(x, weight_t, bias2):
    B, C, H, W = x.shape
    HW = H * W
    Lp = weight_t.shape[1]
    n_label = 1000

    x3 = x.transpose(2, 3, 0, 1).reshape(HW, B, C)

    tb = _choose_tb(B, HW, C, x.dtype.itemsize)
    grid = (B // tb,)

    cost = pl.CostEstimate(
        flops=int(B * C * HW + 2 * B * C * Lp),
        transcendentals=0,
        bytes_accessed=int(x.dtype.itemsize * B * C * HW
                           + 4 * (C * Lp + Lp + B * Lp)))

    out = pl.pallas_call(
        functools.partial(_pool_linear_kernel, inv_hw=float(1.0 / HW)),
        out_shape=jax.ShapeDtypeStruct((B, Lp), jnp.float32),
        grid=grid,
        in_specs=[
            pl.BlockSpec((HW, tb, C), lambda i: (0, i, 0)),
            pl.BlockSpec((C, Lp), lambda i: (0, 0)),
            pl.BlockSpec((1, Lp), lambda i: (0, 0)),
        ],
        out_specs=pl.BlockSpec((tb, Lp), lambda i: (i, 0)),
        compiler_params=pltpu.CompilerParams(
            dimension_semantics=("parallel",),
            vmem_limit_bytes=_VMEM_LIMIT_BYTES),
        cost_estimate=cost,
    )(x3, weight_t, bias2)

    return out[:, :n_label]
